# Initial kernel scaffold; baseline (speedup 1.0000x reference)
import functools

import jax
import jax.numpy as jnp
from jax import lax
from jax.experimental import pallas as pl
from jax.experimental.pallas import tpu as pltpu
from jax.experimental.pallas import tpu_sc as plsc

VOCAB = 1000
HIDDEN = 64
IDS = 50
BATCH = 1024
NC = 2
NS = 16
L = 16
NW = NC * NS
B_PER_W = BATCH // NW
IDS_PER_W = B_PER_W * IDS
NCHUNK = HIDDEN // L


def _embed_body(x_hbm, table_hbm, out_hbm, idx_v, rows_v, out_v, sem):
    wid = lax.axis_index("s") * NC + lax.axis_index("c")
    base = wid * B_PER_W
    pltpu.sync_copy(x_hbm.at[pl.ds(base * IDS, IDS_PER_W)], idx_v)

    def per_row(b, carry):
        pltpu.async_copy(
            table_hbm.at[idx_v.at[pl.ds(b * IDS, IDS)]], rows_v, sem
        ).wait()

        def acc_step(i, accs):
            return tuple(
                a + rows_v[i, pl.ds(c * L, L)] for c, a in enumerate(accs)
            )

        accs = lax.fori_loop(
            0, IDS, acc_step,
            tuple(jnp.zeros((L,), jnp.float32) for _ in range(NCHUNK)),
        )
        scale = jnp.float32(1.0 / IDS)
        for c in range(NCHUNK):
            out_v[b, pl.ds(c * L, L)] = accs[c] * scale
        return carry

    lax.fori_loop(0, B_PER_W, per_row, 0)
    pltpu.sync_copy(out_v, out_hbm.at[pl.ds(base, B_PER_W)])


_embed = functools.partial(
    pl.kernel,
    out_type=jax.ShapeDtypeStruct((BATCH, HIDDEN), jnp.float32),
    mesh=plsc.VectorSubcoreMesh(
        core_axis_name="c", subcore_axis_name="s", num_cores=NC, num_subcores=NS
    ),
    scratch_types=[
        pltpu.VMEM((IDS_PER_W,), jnp.int32),
        pltpu.VMEM((IDS, HIDDEN), jnp.float32),
        pltpu.VMEM((B_PER_W, HIDDEN), jnp.float32),
        pltpu.SemaphoreType.DMA,
    ],
)(_embed_body)


def kernel(x, weights):
    return _embed(x.reshape(-1).astype(jnp.int32), weights)


# speedup vs baseline: 5.1282x; 5.1282x over previous
import functools

import jax
import jax.numpy as jnp
from jax import lax
from jax.experimental import pallas as pl
from jax.experimental.pallas import tpu as pltpu
from jax.experimental.pallas import tpu_sc as plsc

VOCAB = 1000
HIDDEN = 64
IDS = 50
IDS_PAD = 56
BATCH = 1024
NC = 2
NS = 16
L = 16
NW = NC * NS
B_PER_W = BATCH // NW
IDS_PER_W = B_PER_W * IDS_PAD
NCHUNK = HIDDEN // L


def _embed_body(x_hbm, table_hbm, out_hbm, idx_v, rows_v, out_v, sem):
    wid = lax.axis_index("s") * NC + lax.axis_index("c")
    base = wid * B_PER_W
    pltpu.sync_copy(x_hbm.at[pl.ds(base * IDS_PAD, IDS_PER_W)], idx_v)

    def per_row(b, carry):
        pltpu.async_copy(
            table_hbm.at[idx_v.at[pl.ds(b * IDS_PAD, IDS)]], rows_v, sem
        ).wait()

        def acc_step(i, accs):
            return tuple(
                a + rows_v[i, pl.ds(c * L, L)] for c, a in enumerate(accs)
            )

        accs = lax.fori_loop(
            0, IDS, acc_step,
            tuple(jnp.zeros((L,), jnp.float32) for _ in range(NCHUNK)),
        )
        scale = jnp.float32(1.0 / IDS)
        for c in range(NCHUNK):
            out_v[b, pl.ds(c * L, L)] = accs[c] * scale
        return carry

    lax.fori_loop(0, B_PER_W, per_row, 0)
    pltpu.sync_copy(out_v, out_hbm.at[pl.ds(base, B_PER_W)])


_embed = functools.partial(
    pl.kernel,
    out_type=jax.ShapeDtypeStruct((BATCH, HIDDEN), jnp.float32),
    mesh=plsc.VectorSubcoreMesh(
        core_axis_name="c", subcore_axis_name="s", num_cores=NC, num_subcores=NS
    ),
    scratch_types=[
        pltpu.VMEM((IDS_PER_W,), jnp.int32),
        pltpu.VMEM((IDS, HIDDEN), jnp.float32),
        pltpu.VMEM((B_PER_W, HIDDEN), jnp.float32),
        pltpu.SemaphoreType.DMA,
    ],
    compiler_params=pltpu.CompilerParams(use_tc_tiling_on_sc=False),
)(_embed_body)


def kernel(x, weights):
    xp = jnp.pad(x.astype(jnp.int32), ((0, 0), (0, IDS_PAD - IDS)))
    return _embed(xp.reshape(-1), weights)
